# Initial kernel scaffold; baseline (speedup 1.0000x reference)
import functools

import jax
import jax.numpy as jnp
from jax import lax
from jax.experimental import pallas as pl
from jax.experimental.pallas import tpu as pltpu
from jax.experimental.pallas import tpu_sc as plsc

NC = 2
NS = 16
K = 128
G = 32
LANES = 16


def _mesh():
    return plsc.VectorSubcoreMesh(core_axis_name="c", subcore_axis_name="s")


def _make_deg_kernel(n_pad, t_tile):
    rpt = n_pad // NS
    n_chunks = t_tile // K

    @functools.partial(
        pl.kernel,
        mesh=_mesh(),
        out_type=jax.ShapeDtypeStruct((NC, n_pad, LANES), jnp.float32),
        scratch_types=[
            pltpu.VMEM((K,), jnp.int32),
            pltpu.VMEM((K, LANES), jnp.float32),
            pltpu.VMEM_SHARED((n_pad, LANES), jnp.float32),
        ],
    )
    def deg_kernel(dst_hbm, ones_hbm, zeros_hbm, out_hbm, dstv, onesv, deg_sh):
        c = lax.axis_index("c")
        s = lax.axis_index("s")
        wid = c * NS + s
        pltpu.sync_copy(ones_hbm, onesv)
        pltpu.sync_copy(zeros_hbm, deg_sh.at[pl.ds(s * rpt, rpt)])
        plsc.subcore_barrier()
        base = wid * t_tile

        def body(i, carry):
            off = base + i * K
            pltpu.sync_copy(dst_hbm.at[pl.ds(off, K)], dstv)
            pltpu.sync_copy(onesv, deg_sh.at[dstv], add=True)
            return carry

        lax.fori_loop(0, n_chunks, body, 0)
        plsc.subcore_barrier()
        pltpu.sync_copy(deg_sh.at[pl.ds(s * rpt, rpt)],
                        out_hbm.at[c, pl.ds(s * rpt, rpt)])

    return deg_kernel


def _make_scatter_kernel(n, n_pad, d, t_tile):
    rpt = n // NS
    n_chunks = t_tile // K

    @functools.partial(
        pl.kernel,
        mesh=_mesh(),
        out_type=jax.ShapeDtypeStruct((NC, n, d), jnp.float32),
        scratch_types=[
            pltpu.VMEM((K,), jnp.int32),
            pltpu.VMEM((K,), jnp.int32),
            pltpu.VMEM((K, d), jnp.float32),
            pltpu.VMEM_SHARED((n_pad, d), jnp.float32),
            pltpu.SemaphoreType.DMA,
        ],
    )
    def scatter_kernel(g_hbm, src_hbm, dst_hbm, out_hbm,
                       srcv, dstv, rows, acc_sh, sem):
        c = lax.axis_index("c")
        s = lax.axis_index("s")
        wid = c * NS + s
        pltpu.sync_copy(g_hbm.at[pl.ds(s * rpt, rpt)],
                        acc_sh.at[pl.ds(s * rpt, rpt)])
        plsc.subcore_barrier()
        base = wid * t_tile

        def body(i, carry):
            off = base + i * K
            pltpu.sync_copy(src_hbm.at[pl.ds(off, K)], srcv)
            pltpu.sync_copy(dst_hbm.at[pl.ds(off, K)], dstv)
            pltpu.async_copy(g_hbm.at[srcv], rows, sem).wait()
            pltpu.sync_copy(rows, acc_sh.at[dstv], add=True)
            return carry

        lax.fori_loop(0, n_chunks, body, 0)
        plsc.subcore_barrier()
        pltpu.sync_copy(acc_sh.at[pl.ds(s * rpt, rpt)],
                        out_hbm.at[c, pl.ds(s * rpt, rpt)])

    return scatter_kernel


def _prep_body(n):
    def body(degp_ref, x_ref, w_ref, dinv_ref, g_ref):
        dp = degp_ref[0] + degp_ref[1]
        deg = dp[:n, 0:1] + 1.0
        dinv = lax.rsqrt(deg)
        dinv_ref[...] = dinv
        g_ref[...] = jnp.dot(x_ref[...], w_ref[...],
                             preferred_element_type=jnp.float32) * dinv
    return body


def _mid_body(p_ref, g_ref, dinv_ref, b_ref, w_ref, o_ref):
    dinv = dinv_ref[...]
    acc = p_ref[0] + p_ref[1] - g_ref[...]
    h = jnp.maximum(acc * dinv + b_ref[...], 0.0)
    o_ref[...] = jnp.dot(h, w_ref[...],
                         preferred_element_type=jnp.float32) * dinv


def _final_body(p_ref, g_ref, dinv_ref, b_ref, batch_ref, wl_ref, bl_ref, o_ref):
    h = (p_ref[0] + p_ref[1] - g_ref[...]) * dinv_ref[...] + b_ref[...]
    gids = lax.broadcasted_iota(jnp.int32, (1, G), 1)
    onehot = (batch_ref[...] == gids).astype(jnp.float32)
    seg = lax.dot_general(onehot, h, (((0,), (0,)), ((), ())),
                          preferred_element_type=jnp.float32)
    cnt = jnp.sum(onehot, axis=0)[:, None]
    mean = seg / jnp.maximum(cnt, 1.0)
    o_ref[...] = jnp.dot(mean, wl_ref[...],
                         preferred_element_type=jnp.float32) + bl_ref[...]


def kernel(x, edge_index, batch, W1, b1, W2, b2, W3, b3, Wl, bl):
    n, d = x.shape
    e = edge_index.shape[1]
    c_out = Wl.shape[1]
    f32 = jnp.float32

    t_tile = -(-e // (NC * NS * K)) * K
    e_pad = NC * NS * t_tile
    pad = e_pad - e
    src = edge_index[0]
    dst = edge_index[1]
    src_p = jnp.concatenate([src, jnp.zeros((pad,), src.dtype)])
    dst_p = jnp.concatenate([dst, jnp.full((pad,), n, dst.dtype)])

    n_pad_deg = -(-(n + 1) // (NS * LANES)) * (NS * LANES)
    n_pad_acc = n + LANES

    ones_const = jnp.ones((K, LANES), f32)
    zeros_const = jnp.zeros((n_pad_deg // NS, LANES), f32)

    deg_k = _make_deg_kernel(n_pad_deg, t_tile)
    scat_k = _make_scatter_kernel(n, n_pad_acc, d, t_tile)

    degp = deg_k(dst_p, ones_const, zeros_const)

    dinv, g1 = pl.pallas_call(
        _prep_body(n),
        out_shape=(jax.ShapeDtypeStruct((n, 1), f32),
                   jax.ShapeDtypeStruct((n, d), f32)),
    )(degp, x, W1)

    mid = pl.pallas_call(
        _mid_body,
        out_shape=jax.ShapeDtypeStruct((n, d), f32),
    )

    p1 = scat_k(g1, src_p, dst_p)
    g2 = mid(p1, g1, dinv, b1.reshape(1, d), W2)
    p2 = scat_k(g2, src_p, dst_p)
    g3 = mid(p2, g2, dinv, b2.reshape(1, d), W3)
    p3 = scat_k(g3, src_p, dst_p)

    out = pl.pallas_call(
        _final_body,
        out_shape=jax.ShapeDtypeStruct((G, c_out), f32),
    )(p3, g3, dinv, b3.reshape(1, d), batch.reshape(n, 1), Wl,
      bl.reshape(1, c_out))
    return out


# speedup vs baseline: 9.5768x; 9.5768x over previous
import functools

import jax
import jax.numpy as jnp
from jax import lax
from jax.experimental import pallas as pl
from jax.experimental.pallas import tpu as pltpu
from jax.experimental.pallas import tpu_sc as plsc

NC = 2
NS = 16
K = 128
G = 32
LANES = 16


def _mesh():
    return plsc.VectorSubcoreMesh(core_axis_name="c", subcore_axis_name="s")


def _make_deg_kernel(n_pad, d, t_tile):
    rpt = n_pad // NS
    n_chunks = t_tile // K

    @functools.partial(
        pl.kernel,
        mesh=_mesh(),
        out_type=jax.ShapeDtypeStruct((NC, n_pad, d), jnp.float32),
        scratch_types=[
            pltpu.VMEM((K,), jnp.int32),
            pltpu.VMEM((K, d), jnp.float32),
            pltpu.VMEM_SHARED((n_pad, d), jnp.float32),
        ],
    )
    def deg_kernel(dst_hbm, ones_hbm, zeros_hbm, out_hbm, dstv, onesv, deg_sh):
        c = lax.axis_index("c")
        s = lax.axis_index("s")
        wid = c * NS + s
        pltpu.sync_copy(ones_hbm, onesv)
        pltpu.sync_copy(zeros_hbm, deg_sh.at[pl.ds(s * rpt, rpt)])
        plsc.subcore_barrier()
        base = wid * t_tile

        def body(i, carry):
            off = base + i * K
            pltpu.sync_copy(dst_hbm.at[pl.ds(off, K)], dstv)
            pltpu.sync_copy(onesv, deg_sh.at[dstv], add=True)
            return carry

        lax.fori_loop(0, n_chunks, body, 0)
        plsc.subcore_barrier()
        pltpu.sync_copy(deg_sh.at[pl.ds(s * rpt, rpt)],
                        out_hbm.at[c, pl.ds(s * rpt, rpt)])

    return deg_kernel


def _make_scatter_kernel(n_pad, d, t_tile):
    rpt = n_pad // NS
    n_chunks = t_tile // K

    @functools.partial(
        pl.kernel,
        mesh=_mesh(),
        out_type=jax.ShapeDtypeStruct((NC, n_pad, d), jnp.float32),
        scratch_types=[
            pltpu.VMEM((K,), jnp.int32),
            pltpu.VMEM((K,), jnp.int32),
            pltpu.VMEM((K, d), jnp.float32),
            pltpu.VMEM_SHARED((n_pad, d), jnp.float32),
            pltpu.SemaphoreType.DMA,
        ],
    )
    def scatter_kernel(g_hbm, src_hbm, dst_hbm, out_hbm,
                       srcv, dstv, rows, acc_sh, sem):
        c = lax.axis_index("c")
        s = lax.axis_index("s")
        wid = c * NS + s
        pltpu.sync_copy(g_hbm.at[pl.ds(s * rpt, rpt)],
                        acc_sh.at[pl.ds(s * rpt, rpt)])
        plsc.subcore_barrier()
        base = wid * t_tile

        def body(i, carry):
            off = base + i * K
            pltpu.sync_copy(src_hbm.at[pl.ds(off, K)], srcv)
            pltpu.sync_copy(dst_hbm.at[pl.ds(off, K)], dstv)
            pltpu.async_copy(g_hbm.at[srcv], rows, sem).wait()
            pltpu.sync_copy(rows, acc_sh.at[dstv], add=True)
            return carry

        lax.fori_loop(0, n_chunks, body, 0)
        plsc.subcore_barrier()
        pltpu.sync_copy(acc_sh.at[pl.ds(s * rpt, rpt)],
                        out_hbm.at[c, pl.ds(s * rpt, rpt)])

    return scatter_kernel


def _prep_body(n, n_pad):
    def body(degp_ref, x_ref, w_ref, dinv_ref, g_ref):
        deg = degp_ref[0, :, 0:1] + degp_ref[1, :, 0:1] + 1.0
        rowid = lax.broadcasted_iota(jnp.int32, (n_pad, 1), 0)
        dinv = jnp.where(rowid < n, lax.rsqrt(deg), 0.0)
        dinv_ref[...] = dinv
        g_ref[...] = jnp.dot(x_ref[...], w_ref[...],
                             preferred_element_type=jnp.float32) * dinv
    return body


def _mid_body(p_ref, g_ref, dinv_ref, b_ref, w_ref, o_ref):
    dinv = dinv_ref[...]
    acc = p_ref[0] + p_ref[1] - g_ref[...]
    h = jnp.maximum(acc * dinv + b_ref[...], 0.0)
    o_ref[...] = jnp.dot(h, w_ref[...],
                         preferred_element_type=jnp.float32) * dinv


def _final_body(p_ref, g_ref, dinv_ref, b_ref, batch_ref, wl_ref, bl_ref, o_ref):
    h = (p_ref[0] + p_ref[1] - g_ref[...]) * dinv_ref[...] + b_ref[...]
    gids = lax.broadcasted_iota(jnp.int32, (1, G), 1)
    onehot = (batch_ref[...] == gids).astype(jnp.float32)
    seg = lax.dot_general(onehot, h, (((0,), (0,)), ((), ())),
                          preferred_element_type=jnp.float32)
    cnt = jnp.sum(onehot, axis=0)[:, None]
    mean = seg / jnp.maximum(cnt, 1.0)
    o_ref[...] = jnp.dot(mean, wl_ref[...],
                         preferred_element_type=jnp.float32) + bl_ref[...]


def kernel(x, edge_index, batch, W1, b1, W2, b2, W3, b3, Wl, bl):
    n, d = x.shape
    e = edge_index.shape[1]
    c_out = Wl.shape[1]
    f32 = jnp.float32

    t_tile = -(-e // (NC * NS * K)) * K
    e_pad = NC * NS * t_tile
    pad = e_pad - e
    src = edge_index[0]
    dst = edge_index[1]
    src_p = jnp.concatenate([src, jnp.zeros((pad,), src.dtype)])
    dst_p = jnp.concatenate([dst, jnp.full((pad,), n, dst.dtype)])

    n_pad = -(-n // (NS * 8)) * (NS * 8)
    if n_pad == n:
        n_pad += NS * 8
    ones_const = jnp.ones((K, d), f32)
    zeros_const = jnp.zeros((n_pad // NS, d), f32)
    x_p = jnp.concatenate([x, jnp.zeros((n_pad - n, d), f32)])
    batch_p = jnp.concatenate([batch, jnp.full((n_pad - n,), G, batch.dtype)])

    deg_k = _make_deg_kernel(n_pad, d, t_tile)
    scat_k = _make_scatter_kernel(n_pad, d, t_tile)

    degp = deg_k(dst_p, ones_const, zeros_const)

    dinv, g1 = pl.pallas_call(
        _prep_body(n, n_pad),
        out_shape=(jax.ShapeDtypeStruct((n_pad, 1), f32),
                   jax.ShapeDtypeStruct((n_pad, d), f32)),
    )(degp, x_p, W1)

    mid = pl.pallas_call(
        _mid_body,
        out_shape=jax.ShapeDtypeStruct((n_pad, d), f32),
    )

    p1 = scat_k(g1, src_p, dst_p)
    g2 = mid(p1, g1, dinv, b1.reshape(1, d), W2)
    p2 = scat_k(g2, src_p, dst_p)
    g3 = mid(p2, g2, dinv, b2.reshape(1, d), W3)
    p3 = scat_k(g3, src_p, dst_p)

    out = pl.pallas_call(
        _final_body,
        out_shape=jax.ShapeDtypeStruct((G, c_out), f32),
    )(p3, g3, dinv, b3.reshape(1, d), batch_p.reshape(n_pad, 1), Wl,
      bl.reshape(1, c_out))
    return out
